# baseline (speedup 1.0000x reference)
import functools

import jax
import jax.numpy as jnp
from jax.experimental import pallas as pl
from jax.experimental.pallas import tpu as pltpu

_ALPHA = 0.2


def _leaky(x):
    return jnp.where(x > 0, x, _ALPHA * x)


def _gcn_cast_body(adj_ref, y_ref, out_ref, cache_ref):
    ab = adj_ref[...].astype(jnp.bfloat16)
    cache_ref[...] = ab
    acc = jnp.dot(ab, y_ref[...], preferred_element_type=jnp.float32)
    out_ref[...] = _leaky(acc)


def _gcn_body(adj_ref, y_ref, out_ref):
    acc = jnp.dot(adj_ref[...], y_ref[...], preferred_element_type=jnp.float32)
    out_ref[...] = _leaky(acc)


def _gcn_cast(adj, y, block_rows):
    n, k = adj.shape
    d = y.shape[1]
    grid = (pl.cdiv(n, block_rows),)
    out, cache = pl.pallas_call(
        _gcn_cast_body,
        grid=grid,
        in_specs=[
            pl.BlockSpec((block_rows, k), lambda i: (i, 0)),
            pl.BlockSpec((k, d), lambda i: (0, 0)),
        ],
        out_specs=(
            pl.BlockSpec((block_rows, d), lambda i: (i, 0)),
            pl.BlockSpec((block_rows, k), lambda i: (i, 0)),
        ),
        out_shape=(
            jax.ShapeDtypeStruct((n, d), jnp.float32),
            jax.ShapeDtypeStruct((n, k), jnp.bfloat16),
        ),
        compiler_params=pltpu.CompilerParams(
            dimension_semantics=("arbitrary",),
        ),
    )(adj, y)
    return out, cache


def _gcn(adj_bf16, y, block_rows):
    n, k = adj_bf16.shape
    d = y.shape[1]
    grid = (pl.cdiv(n, block_rows),)
    return pl.pallas_call(
        _gcn_body,
        grid=grid,
        in_specs=[
            pl.BlockSpec((block_rows, k), lambda i: (i, 0)),
            pl.BlockSpec((k, d), lambda i: (0, 0)),
        ],
        out_specs=pl.BlockSpec((block_rows, d), lambda i: (i, 0)),
        out_shape=jax.ShapeDtypeStruct((n, d), jnp.float32),
        compiler_params=pltpu.CompilerParams(
            dimension_semantics=("arbitrary",),
        ),
    )(adj_bf16, y)


def _mm_body(x_ref, w_ref, o_ref):
    o_ref[...] = jnp.dot(
        x_ref[...], w_ref[...], preferred_element_type=jnp.float32
    ).astype(o_ref.dtype)


def _mm_bf16(x, w, block_rows=2000):
    n, d = x.shape
    grid = (pl.cdiv(n, block_rows),)
    return pl.pallas_call(
        _mm_body,
        grid=grid,
        in_specs=[
            pl.BlockSpec((block_rows, d), lambda i: (i, 0)),
            pl.BlockSpec(w.shape, lambda i: (0, 0)),
        ],
        out_specs=pl.BlockSpec((block_rows, w.shape[1]), lambda i: (i, 0)),
        out_shape=jax.ShapeDtypeStruct((n, w.shape[1]), jnp.bfloat16),
    )(x, w)


def _lin2_body(act, a_ref, b_ref, wa_ref, wb_ref, bias_ref, o_ref):
    acc = jnp.dot(a_ref[...], wa_ref[...], preferred_element_type=jnp.float32)
    acc = acc + jnp.dot(b_ref[...], wb_ref[...], preferred_element_type=jnp.float32)
    acc = acc + bias_ref[...]
    if act:
        acc = _leaky(acc)
    o_ref[...] = acc


def _lin2(a, b, W, bias, act, block_rows=2000):
    n, d = a.shape
    wt = W.T
    wa, wb = wt[:d], wt[d:]
    bias2 = bias.reshape(1, -1)
    grid = (pl.cdiv(n, block_rows),)
    return pl.pallas_call(
        functools.partial(_lin2_body, act),
        grid=grid,
        in_specs=[
            pl.BlockSpec((block_rows, d), lambda i: (i, 0)),
            pl.BlockSpec((block_rows, d), lambda i: (i, 0)),
            pl.BlockSpec(wa.shape, lambda i: (0, 0)),
            pl.BlockSpec(wb.shape, lambda i: (0, 0)),
            pl.BlockSpec(bias2.shape, lambda i: (0, 0)),
        ],
        out_specs=pl.BlockSpec((block_rows, d), lambda i: (i, 0)),
        out_shape=jax.ShapeDtypeStruct((n, d), jnp.float32),
    )(a, b, wa, wb, bias2)


_R_CAST = 128
_R_BF16 = 256


def kernel(ufea, vfea, UV_adj, VU_adj, UU_adj, VV_adj, params):
    p = params
    u, v = ufea, vfea

    User_ho1, UVb = _gcn_cast(UV_adj, _mm_bf16(v, p['l0_gc3']), _R_CAST)
    Item_ho1, VUb = _gcn_cast(VU_adj, _mm_bf16(u, p['l0_gc4']), _R_CAST)
    User_ho, uub = _gcn_cast(UU_adj, _mm_bf16(u, p['l0_gc1']), _R_CAST)
    Item_ho, vvb = _gcn_cast(VV_adj, _mm_bf16(v, p['l0_gc2']), _R_CAST)

    User_ho = _lin2(User_ho, User_ho1, p['l0_user_union1_W'], p['l0_user_union1_b'], act=False)
    Item_ho = _lin2(Item_ho, Item_ho1, p['l0_item_union1_W'], p['l0_item_union1_b'], act=False)
    u1 = _lin2(User_ho, u, p['l0_user_union_W'], p['l0_user_union_b'], act=True)
    v1 = _lin2(Item_ho, v, p['l0_item_union_W'], p['l0_item_union_b'], act=True)

    Uho1 = _gcn(UVb, _mm_bf16(v1, p['ll_gc3']), _R_BF16)
    Iho1 = _gcn(VUb, _mm_bf16(u1, p['ll_gc4']), _R_BF16)
    Uho = _gcn(uub, _mm_bf16(u1, p['ll_gc1']), _R_BF16)
    Iho = _gcn(vvb, _mm_bf16(v1, p['ll_gc2']), _R_BF16)

    Uho = _lin2(Uho, Uho1, p['ll_user_union_W'], p['ll_user_union_b'], act=False)
    Iho = _lin2(Iho, Iho1, p['ll_item_union_W'], p['ll_item_union_b'], act=False)

    Uho = _gcn(uub, _mm_bf16(Uho, p['ll_gc5']), _R_BF16)
    Iho = _gcn(vvb, _mm_bf16(Iho, p['ll_gc6']), _R_BF16)
    Umean = _gcn(uub, _mm_bf16(Uho, p['ll_gc3_mean']), _R_BF16)
    Imean = _gcn(vvb, _mm_bf16(Iho, p['ll_gc4_mean']), _R_BF16)

    user = _lin2(Umean, u1, p['ll_user_union_1_W'], p['ll_user_union_1_b'], act=False)
    item = _lin2(Imean, v1, p['ll_item_union_1_W'], p['ll_item_union_1_b'], act=False)
    return (user, item)


# speedup vs baseline: 1.1797x; 1.1797x over previous
import jax
import jax.numpy as jnp
from jax.experimental import pallas as pl
from jax.experimental.pallas import tpu as pltpu

_ALPHA = 0.2
_R_CAST = 256
_R_BF16 = 512


def _leaky(x):
    return jnp.where(x > 0, x, _ALPHA * x)


def _dot(a, b):
    return jnp.dot(a, b, preferred_element_type=jnp.float32)


def _make_body(cast, n_ep, act_end):
    def body(*refs):
        y_scr = refs[-1]
        if cast:
            out_ref, cache_ref = refs[-3], refs[-2]
            ins = refs[:-3]
        else:
            out_ref = refs[-2]
            ins = refs[:-2]
        adj_ref, x_ref, w_ref = ins[:3]
        ep_refs = ins[3:]

        @pl.when(pl.program_id(0) == 0)
        def _():
            y_scr[...] = _dot(x_ref[...], w_ref[...]).astype(jnp.bfloat16)

        a = adj_ref[...]
        if cast:
            a = a.astype(jnp.bfloat16)
            cache_ref[...] = a
        g = _leaky(_dot(a, y_scr[...]))
        for j in range(n_ep):
            e, wa, wb, bias = ep_refs[4 * j:4 * j + 4]
            g = _dot(g, wa[...]) + _dot(e[...], wb[...]) + bias[...]
        if act_end:
            g = _leaky(g)
        out_ref[...] = g
    return body


def _fused_pass(adj, x, w, *, cast, block_rows, ep=(), act_end=False):
    n, k = adj.shape
    d = x.shape[1]
    const = lambda i: (0, 0)
    row = lambda i: (i, 0)
    ins = [adj, x, w]
    in_specs = [
        pl.BlockSpec((block_rows, k), row),
        pl.BlockSpec((k, d), const),
        pl.BlockSpec(w.shape, const),
    ]
    for e, wa, wb, bias in ep:
        ins += [e, wa, wb, bias]
        in_specs += [
            pl.BlockSpec((block_rows, d), row),
            pl.BlockSpec(wa.shape, const),
            pl.BlockSpec(wb.shape, const),
            pl.BlockSpec(bias.shape, const),
        ]
    out_shape = [jax.ShapeDtypeStruct((n, d), jnp.float32)]
    out_specs = [pl.BlockSpec((block_rows, d), row)]
    if cast:
        out_shape.append(jax.ShapeDtypeStruct((n, k), jnp.bfloat16))
        out_specs.append(pl.BlockSpec((block_rows, k), row))
    res = pl.pallas_call(
        _make_body(cast, len(ep), act_end),
        grid=(pl.cdiv(n, block_rows),),
        in_specs=in_specs,
        out_specs=tuple(out_specs),
        out_shape=tuple(out_shape),
        scratch_shapes=[pltpu.VMEM((k, d), jnp.bfloat16)],
        compiler_params=pltpu.CompilerParams(
            dimension_semantics=("arbitrary",),
        ),
    )(*ins)
    return res if cast else res[0]


def kernel(ufea, vfea, UV_adj, VU_adj, UU_adj, VV_adj, params):
    p = params
    u, v = ufea, vfea
    d = u.shape[1]

    def split(name):
        wt = p[name + '_W'].T
        return wt[:d], wt[d:], p[name + '_b'].reshape(1, d)

    User_ho1, UVb = _fused_pass(UV_adj, v, p['l0_gc3'], cast=True, block_rows=_R_CAST)
    Item_ho1, VUb = _fused_pass(VU_adj, u, p['l0_gc4'], cast=True, block_rows=_R_CAST)
    u1, uub = _fused_pass(
        UU_adj, u, p['l0_gc1'], cast=True, block_rows=_R_CAST,
        ep=[(User_ho1,) + split('l0_user_union1'), (u,) + split('l0_user_union')],
        act_end=True)
    v1, vvb = _fused_pass(
        VV_adj, v, p['l0_gc2'], cast=True, block_rows=_R_CAST,
        ep=[(Item_ho1,) + split('l0_item_union1'), (v,) + split('l0_item_union')],
        act_end=True)

    Uho1 = _fused_pass(UVb, v1, p['ll_gc3'], cast=False, block_rows=_R_BF16)
    Iho1 = _fused_pass(VUb, u1, p['ll_gc4'], cast=False, block_rows=_R_BF16)
    Uho2 = _fused_pass(uub, u1, p['ll_gc1'], cast=False, block_rows=_R_BF16,
                       ep=[(Uho1,) + split('ll_user_union')])
    Iho2 = _fused_pass(vvb, v1, p['ll_gc2'], cast=False, block_rows=_R_BF16,
                       ep=[(Iho1,) + split('ll_item_union')])
    Uho3 = _fused_pass(uub, Uho2, p['ll_gc5'], cast=False, block_rows=_R_BF16)
    Iho3 = _fused_pass(vvb, Iho2, p['ll_gc6'], cast=False, block_rows=_R_BF16)
    user = _fused_pass(uub, Uho3, p['ll_gc3_mean'], cast=False, block_rows=_R_BF16,
                       ep=[(u1,) + split('ll_user_union_1')])
    item = _fused_pass(vvb, Iho3, p['ll_gc4_mean'], cast=False, block_rows=_R_BF16,
                       ep=[(v1,) + split('ll_item_union_1')])
    return (user, item)
